# baseline (speedup 1.0000x reference)
import functools

import jax
import jax.numpy as jnp
from jax.experimental import pallas as pl
from jax.experimental.pallas import tpu as pltpu

N = 10000
D = 128
H = 256
C = 64
E = 320000

EBLK = 8192
EPAD = ((E + EBLK - 1) // EBLK) * EBLK

_NEG = -3.0e38


def _seg_max_body(src_ref, dst_ref, x_ref, agg_ref):
    step = pl.program_id(0)

    @pl.when(step == 0)
    def _init():
        agg_ref[...] = jnp.full(agg_ref.shape, _NEG, jnp.float32)

    def body(i, carry):
        s = src_ref[i]
        d = dst_ref[i]
        row = x_ref[s, :]
        agg_ref[d, :] = jnp.maximum(agg_ref[d, :], row)
        return carry

    jax.lax.fori_loop(0, EBLK, body, 0)


def _seg_max(x, src, dst):
    n, d = x.shape
    return pl.pallas_call(
        _seg_max_body,
        grid=(EPAD // EBLK,),
        in_specs=[
            pl.BlockSpec((EBLK,), lambda i: (i,), memory_space=pltpu.SMEM),
            pl.BlockSpec((EBLK,), lambda i: (i,), memory_space=pltpu.SMEM),
            pl.BlockSpec((n, d), lambda i: (0, 0)),
        ],
        out_specs=pl.BlockSpec((n, d), lambda i: (0, 0)),
        out_shape=jax.ShapeDtypeStruct((n, d), jnp.float32),
    )(src, dst, x)


def _mm_body(agg_ref, x_ref, wl_ref, b_ref, wr_ref, o_ref, *, relu):
    a = agg_ref[...]
    a = jnp.where(a <= _NEG, 0.0, a)
    o = (jnp.dot(a, wl_ref[...], preferred_element_type=jnp.float32)
         + b_ref[...][None, :]
         + jnp.dot(x_ref[...], wr_ref[...], preferred_element_type=jnp.float32))
    if relu:
        o = jnp.maximum(o, 0.0)
    o_ref[...] = o


def _mm(agg, x, wl, b, wr, relu):
    n, din = x.shape
    dout = wl.shape[1]
    rblk = 2000
    return pl.pallas_call(
        functools.partial(_mm_body, relu=relu),
        grid=(n // rblk,),
        in_specs=[
            pl.BlockSpec((rblk, din), lambda i: (i, 0)),
            pl.BlockSpec((rblk, din), lambda i: (i, 0)),
            pl.BlockSpec((din, dout), lambda i: (0, 0)),
            pl.BlockSpec((dout,), lambda i: (0,)),
            pl.BlockSpec((din, dout), lambda i: (0, 0)),
        ],
        out_specs=pl.BlockSpec((rblk, dout), lambda i: (i, 0)),
        out_shape=jax.ShapeDtypeStruct((n, dout), jnp.float32),
    )(agg, x, wl, b, wr)


def kernel(x, edge_index, W1_l, b1_l, W1_r, W2_l, b2_l, W2_r):
    pad = EPAD - E
    src = jnp.concatenate([edge_index[0], jnp.broadcast_to(edge_index[0, -1:], (pad,))])
    dst = jnp.concatenate([edge_index[1], jnp.broadcast_to(edge_index[1, -1:], (pad,))])
    agg1 = _seg_max(x, src, dst)
    h = _mm(agg1, x, W1_l, b1_l, W1_r, relu=True)
    agg2 = _seg_max(h, src, dst)
    out = _mm(agg2, h, W2_l, b2_l, W2_r, relu=False)
    return out


# speedup vs baseline: 1.6601x; 1.6601x over previous
import functools

import jax
import jax.numpy as jnp
from jax import lax
from jax.experimental import pallas as pl
from jax.experimental.pallas import tpu as pltpu
from jax.experimental.pallas import tpu_sc as plsc

N = 10000
D = 128
H = 256
C = 64
E = 320000

NC = 2
NS = 16
NW = NC * NS

PB = 320
N2 = NW * PB
W = 2048
RB = 128
EPAD = ((E + W - 1) // W) * W

_NEG = -3.0e38


def _seg_max_sc_body(Df, x_hbm, src_hbm, dst_hbm, out_hbm,
                     dbuf, sbuf, mpk, msrc, mdl, rowbuf, acc, sem):
    wid = lax.axis_index("s") * NC + lax.axis_index("c")
    lo = wid * PB
    lo16 = jnp.broadcast_to(lo, (16,))
    hi16 = lo16 + PB
    neg = jnp.full((16,), _NEG, jnp.float32)
    iota = lax.iota(jnp.int32, 16)
    pad_rows = wid * 16 + iota
    nwin = src_hbm.shape[0] // W

    def init_body(r, _):
        for c in range(Df // 16):
            acc[r, pl.ds(16 * c, 16)] = neg
        return 0

    lax.fori_loop(0, PB, init_body, 0)

    def win_body(w, _):
        pltpu.sync_copy(dst_hbm.at[pl.ds(w * W, W)], dbuf)
        pltpu.sync_copy(src_hbm.at[pl.ds(w * W, W)], sbuf)

        def scan_body(v, cnt):
            sl = pl.ds(16 * v, 16)
            d16 = dbuf[sl]
            s16 = sbuf[sl]
            mask = (d16 >= lo16) & (d16 < hi16)
            val = s16 * 512 + (d16 - lo16)
            _, sv = plsc.sort_key_val(jnp.where(mask, 0, 1), val)
            mpk[pl.ds(cnt, 16)] = sv
            return cnt + plsc.all_reduce_population_count(mask)[0]

        cnt = lax.fori_loop(0, W // 16, scan_body, jnp.int32(0))

        @pl.when(cnt > 0)
        def _process():
            nch = (cnt + RB - 1) // RB

            def unpack_body(u, _):
                gidx = 16 * u + iota
                p = mpk[pl.ds(16 * u, 16)]
                valid = gidx < cnt
                msrc[pl.ds(16 * u, 16)] = jnp.where(
                    valid, lax.shift_right_logical(p, 9), pad_rows)
                mdl[pl.ds(16 * u, 16)] = p & 511
                return 0

            lax.fori_loop(0, (nch * RB) // 16, unpack_body, 0)

            def chunk_body(j, _):
                base = j * RB
                cp = pltpu.async_copy(x_hbm.at[msrc.at[pl.ds(base, RB)]],
                                      rowbuf, sem)
                cp.wait()
                lim = jnp.minimum(cnt - base, RB)

                def edge_body(e, _):
                    dl = mdl[pl.ds(base + e, 16)][0]
                    for c in range(Df // 16):
                        slc = pl.ds(16 * c, 16)
                        acc[dl, slc] = jnp.maximum(acc[dl, slc],
                                                   rowbuf[e, slc])
                    return 0

                lax.fori_loop(0, lim, edge_body, 0)
                return 0

            lax.fori_loop(0, nch, chunk_body, 0)

        return 0

    lax.fori_loop(0, nwin, win_body, 0)
    pltpu.sync_copy(acc, out_hbm.at[pl.ds(lo, PB)])


def _seg_max_sc(x, src, dst):
    n, Df = x.shape
    mesh = plsc.VectorSubcoreMesh(core_axis_name="c", subcore_axis_name="s")
    kfn = pl.kernel(
        functools.partial(_seg_max_sc_body, Df),
        mesh=mesh,
        compiler_params=pltpu.CompilerParams(needs_layout_passes=False),
        out_type=jax.ShapeDtypeStruct((N2, Df), jnp.float32),
        scratch_types=[
            pltpu.VMEM((W,), jnp.int32),
            pltpu.VMEM((W,), jnp.int32),
            pltpu.VMEM((W + 16,), jnp.int32),
            pltpu.VMEM((W,), jnp.int32),
            pltpu.VMEM((W + 16,), jnp.int32),
            pltpu.VMEM((RB, Df), jnp.float32),
            pltpu.VMEM((PB, Df), jnp.float32),
            pltpu.SemaphoreType.DMA,
        ],
    )
    return kfn(x, src, dst)


def _mm_body(agg_ref, x_ref, wl_ref, b_ref, wr_ref, o_ref, *, relu):
    a = agg_ref[...]
    a = jnp.where(a <= _NEG, 0.0, a)
    o = (jnp.dot(a, wl_ref[...], preferred_element_type=jnp.float32)
         + b_ref[...][None, :]
         + jnp.dot(x_ref[...], wr_ref[...], preferred_element_type=jnp.float32))
    if relu:
        o = jnp.maximum(o, 0.0)
    o_ref[...] = o


def _mm(agg, x, wl, b, wr, relu):
    n, din = x.shape
    dout = wl.shape[1]
    rblk = 2000
    return pl.pallas_call(
        functools.partial(_mm_body, relu=relu),
        grid=(n // rblk,),
        in_specs=[
            pl.BlockSpec((rblk, din), lambda i: (i, 0)),
            pl.BlockSpec((rblk, din), lambda i: (i, 0)),
            pl.BlockSpec((din, dout), lambda i: (0, 0)),
            pl.BlockSpec((dout,), lambda i: (0,)),
            pl.BlockSpec((din, dout), lambda i: (0, 0)),
        ],
        out_specs=pl.BlockSpec((rblk, dout), lambda i: (i, 0)),
        out_shape=jax.ShapeDtypeStruct((n, dout), jnp.float32),
    )(agg, x, wl, b, wr)


def kernel(x, edge_index, W1_l, b1_l, W1_r, W2_l, b2_l, W2_r):
    pad = EPAD - E
    src = jnp.concatenate([edge_index[0], jnp.broadcast_to(edge_index[0, -1:], (pad,))])
    dst = jnp.concatenate([edge_index[1], jnp.broadcast_to(edge_index[1, -1:], (pad,))])
    agg1 = _seg_max_sc(x, src, dst)[:N]
    h = _mm(agg1, x, W1_l, b1_l, W1_r, relu=True)
    agg2 = _seg_max_sc(h, src, dst)[:N]
    out = _mm(agg2, h, W2_l, b2_l, W2_r, relu=False)
    return out
